# Initial kernel scaffold; baseline (speedup 1.0000x reference)
import jax
import jax.numpy as jnp
from jax.experimental import pallas as pl


def kernel(node_embedding, embedding_table_weight, batch, sequence, itemset_len, sequence_len, cue, W1_w, W1_b, W2_w, W2_b, q_w, q_b, W3_w, W3_b):
    raise NotImplementedError("write your pallas kernel here")


# speedup vs baseline: 3.9307x; 3.9307x over previous
import jax
import jax.numpy as jnp
from jax import lax
from jax.experimental import pallas as pl

B = 1024
N_PER = 32
H = 128
L = 16
PAD = 8
SB = 8
R = SB * L
NODES = SB * N_PER
CW = 2048


def _attn_kernel(nodes_ref, seq_ref, il_ref, w1_ref, b1_ref, w2_ref, b2_ref,
                 q_ref, qb_ref, w3_ref, b3_ref, sh_ref):
    nodes = nodes_ref[...]
    seq = seq_ref[...]
    il = il_ref[...]

    base = (lax.broadcasted_iota(jnp.int32, (R, NODES), 0) >> 4) << 5
    cols = lax.broadcasted_iota(jnp.int32, (R, NODES), 1)
    counts = jnp.zeros((R, NODES), jnp.float32)
    for p in range(PAD):
        sp = seq[:, p:p + 1]
        t = jnp.where(sp < N_PER, sp, 100000) + base
        counts = counts + (cols == t).astype(jnp.float32)

    sess_sum = jnp.dot(counts, nodes, preferred_element_type=jnp.float32)
    sess = sess_sum / il

    sel_r = lax.broadcasted_iota(jnp.int32, (SB, R), 1)
    sel_s = lax.broadcasted_iota(jnp.int32, (SB, R), 0) * L + (L - 1)
    sel = (sel_r == sel_s).astype(jnp.float32)
    v_n = jnp.dot(sel, sess, preferred_element_type=jnp.float32)

    rep = ((lax.broadcasted_iota(jnp.int32, (R, SB), 0) >> 4) ==
           lax.broadcasted_iota(jnp.int32, (R, SB), 1)).astype(jnp.float32)
    u1 = lax.dot_general(v_n, w1_ref[...], (((1,), (1,)), ((), ())),
                         preferred_element_type=jnp.float32) + b1_ref[...]
    t1 = jnp.dot(rep, u1, preferred_element_type=jnp.float32)
    t2 = lax.dot_general(sess, w2_ref[...], (((1,), (1,)), ((), ())),
                         preferred_element_type=jnp.float32) + b2_ref[...]
    a = jax.nn.sigmoid(t1 + t2)
    alpha_b = lax.dot_general(a, q_ref[...], (((1,), (0,)), ((), ())),
                              preferred_element_type=jnp.float32) + qb_ref[...]
    s_g = lax.dot_general(rep, alpha_b * sess, (((0,), (0,)), ((), ())),
                          preferred_element_type=jnp.float32)
    w3 = w3_ref[...]
    s_h = (lax.dot_general(v_n, w3[:, :H], (((1,), (1,)), ((), ())),
                           preferred_element_type=jnp.float32) +
           lax.dot_general(s_g, w3[:, H:], (((1,), (1,)), ((), ())),
                           preferred_element_type=jnp.float32) + b3_ref[...])
    sh_ref[...] = s_h


def _score_kernel(sh_ref, e_ref, cue_ref, out_ref, y_ref):
    j = pl.program_id(0)
    sh = sh_ref[...]
    eb = e_ref[...]
    scores = lax.dot_general(sh, eb, (((1,), (1,)), ((), ())),
                             preferred_element_type=jnp.float32)
    out_ref[...] = scores
    col = lax.broadcasted_iota(jnp.int32, (B, CW), 1) + j * CW
    hit = jnp.where(col == cue_ref[...], scores, 0.0)
    y_part = jnp.sum(hit, axis=1, keepdims=True)

    @pl.when(j == 0)
    def _():
        y_ref[...] = jnp.zeros_like(y_ref)

    y_ref[...] += y_part


def kernel(node_embedding, embedding_table_weight, batch, sequence, itemset_len,
           sequence_len, cue, W1_w, W1_b, W2_w, W2_b, q_w, q_b, W3_w, W3_b):
    vocab = embedding_table_weight.shape[0]
    il_f = itemset_len.astype(jnp.float32).reshape(B * L, 1)
    cue2 = cue.reshape(B, 1)

    n_blocks = B // SB
    s_h = pl.pallas_call(
        _attn_kernel,
        grid=(n_blocks,),
        in_specs=[
            pl.BlockSpec((NODES, H), lambda i: (i, 0)),
            pl.BlockSpec((R, PAD), lambda i: (i, 0)),
            pl.BlockSpec((R, 1), lambda i: (i, 0)),
            pl.BlockSpec((H, H), lambda i: (0, 0)),
            pl.BlockSpec((1, H), lambda i: (0, 0)),
            pl.BlockSpec((H, H), lambda i: (0, 0)),
            pl.BlockSpec((1, H), lambda i: (0, 0)),
            pl.BlockSpec((H, H), lambda i: (0, 0)),
            pl.BlockSpec((1, H), lambda i: (0, 0)),
            pl.BlockSpec((H, 2 * H), lambda i: (0, 0)),
            pl.BlockSpec((1, H), lambda i: (0, 0)),
        ],
        out_specs=pl.BlockSpec((SB, H), lambda i: (i, 0)),
        out_shape=jax.ShapeDtypeStruct((B, H), jnp.float32),
    )(node_embedding, sequence, il_f, W1_w, W1_b.reshape(1, H), W2_w,
      W2_b.reshape(1, H), jnp.broadcast_to(q_w.reshape(H, 1), (H, H)),
      jnp.broadcast_to(q_b.reshape(1, 1), (1, H)), W3_w, W3_b.reshape(1, H))

    n_vblocks = pl.cdiv(vocab, CW)
    all_scores, y_hat = pl.pallas_call(
        _score_kernel,
        grid=(n_vblocks,),
        in_specs=[
            pl.BlockSpec((B, H), lambda j: (0, 0)),
            pl.BlockSpec((CW, H), lambda j: (j, 0)),
            pl.BlockSpec((B, 1), lambda j: (0, 0)),
        ],
        out_specs=[
            pl.BlockSpec((B, CW), lambda j: (0, j)),
            pl.BlockSpec((B, 1), lambda j: (0, 0)),
        ],
        out_shape=[
            jax.ShapeDtypeStruct((B, vocab), jnp.float32),
            jax.ShapeDtypeStruct((B, 1), jnp.float32),
        ],
    )(s_h, embedding_table_weight, cue2)

    return (y_hat.reshape(B), all_scores)


# speedup vs baseline: 4.6403x; 1.1805x over previous
import jax
import jax.numpy as jnp
from jax import lax
from jax.experimental import pallas as pl

B = 1024
N_PER = 32
H = 128
L = 16
PAD = 8
SB = 64
SUB = 8
R = SB * L
RS = SUB * L
NS = SUB * N_PER
CW = 2048


def _attn_kernel(nodes_ref, seq_ref, il_ref, w1_ref, b1_ref, w2_ref, b2_ref,
                 q_ref, qb_ref, w3_ref, b3_ref, sh_ref):
    il = il_ref[...]

    base = (lax.broadcasted_iota(jnp.int32, (RS, NS), 0) >> 4) << 5
    cols = lax.broadcasted_iota(jnp.int32, (RS, NS), 1)
    parts = []
    for g in range(SB // SUB):
        seq_g = seq_ref[g * RS:(g + 1) * RS, :]
        counts = jnp.zeros((RS, NS), jnp.float32)
        for p in range(PAD):
            sp = seq_g[:, p:p + 1]
            t = jnp.where(sp < N_PER, sp, 100000) + base
            counts = counts + (cols == t).astype(jnp.float32)
        parts.append(jnp.dot(counts, nodes_ref[g * NS:(g + 1) * NS, :],
                             preferred_element_type=jnp.float32))
    sess_sum = jnp.concatenate(parts, axis=0)
    sess = sess_sum / il

    sel_r = lax.broadcasted_iota(jnp.int32, (SB, R), 1)
    sel_s = lax.broadcasted_iota(jnp.int32, (SB, R), 0) * L + (L - 1)
    sel = (sel_r == sel_s).astype(jnp.float32)
    v_n = jnp.dot(sel, sess, preferred_element_type=jnp.float32)

    rep = ((lax.broadcasted_iota(jnp.int32, (R, SB), 0) >> 4) ==
           lax.broadcasted_iota(jnp.int32, (R, SB), 1)).astype(jnp.float32)
    u1 = lax.dot_general(v_n, w1_ref[...], (((1,), (1,)), ((), ())),
                         preferred_element_type=jnp.float32) + b1_ref[...]
    t1 = jnp.dot(rep, u1, preferred_element_type=jnp.float32)
    t2 = lax.dot_general(sess, w2_ref[...], (((1,), (1,)), ((), ())),
                         preferred_element_type=jnp.float32) + b2_ref[...]
    a = jax.nn.sigmoid(t1 + t2)
    alpha_b = lax.dot_general(a, q_ref[...], (((1,), (0,)), ((), ())),
                              preferred_element_type=jnp.float32) + qb_ref[...]
    s_g = lax.dot_general(rep, alpha_b * sess, (((0,), (0,)), ((), ())),
                          preferred_element_type=jnp.float32)
    w3 = w3_ref[...]
    s_h = (lax.dot_general(v_n, w3[:, :H], (((1,), (1,)), ((), ())),
                           preferred_element_type=jnp.float32) +
           lax.dot_general(s_g, w3[:, H:], (((1,), (1,)), ((), ())),
                           preferred_element_type=jnp.float32) + b3_ref[...])
    sh_ref[...] = s_h


def _score_kernel(sh_ref, e_ref, cue_ref, out_ref, y_ref):
    j = pl.program_id(0)
    sh = sh_ref[...]
    eb = e_ref[...]
    scores = lax.dot_general(sh, eb, (((1,), (1,)), ((), ())),
                             preferred_element_type=jnp.float32)
    out_ref[...] = scores
    col = lax.broadcasted_iota(jnp.int32, (B, CW), 1) + j * CW
    hit = jnp.where(col == cue_ref[...], scores, 0.0)
    y_part = jnp.sum(hit, axis=1, keepdims=True)

    @pl.when(j == 0)
    def _():
        y_ref[...] = jnp.zeros_like(y_ref)

    y_ref[...] += y_part


def kernel(node_embedding, embedding_table_weight, batch, sequence, itemset_len,
           sequence_len, cue, W1_w, W1_b, W2_w, W2_b, q_w, q_b, W3_w, W3_b):
    vocab = embedding_table_weight.shape[0]
    il_f = itemset_len.astype(jnp.float32).reshape(B * L, 1)
    cue2 = cue.reshape(B, 1)

    n_blocks = B // SB
    s_h = pl.pallas_call(
        _attn_kernel,
        grid=(n_blocks,),
        in_specs=[
            pl.BlockSpec((SB * N_PER, H), lambda i: (i, 0)),
            pl.BlockSpec((R, PAD), lambda i: (i, 0)),
            pl.BlockSpec((R, 1), lambda i: (i, 0)),
            pl.BlockSpec((H, H), lambda i: (0, 0)),
            pl.BlockSpec((1, H), lambda i: (0, 0)),
            pl.BlockSpec((H, H), lambda i: (0, 0)),
            pl.BlockSpec((1, H), lambda i: (0, 0)),
            pl.BlockSpec((H, H), lambda i: (0, 0)),
            pl.BlockSpec((1, H), lambda i: (0, 0)),
            pl.BlockSpec((H, 2 * H), lambda i: (0, 0)),
            pl.BlockSpec((1, H), lambda i: (0, 0)),
        ],
        out_specs=pl.BlockSpec((SB, H), lambda i: (i, 0)),
        out_shape=jax.ShapeDtypeStruct((B, H), jnp.float32),
    )(node_embedding, sequence, il_f, W1_w, W1_b.reshape(1, H), W2_w,
      W2_b.reshape(1, H), jnp.broadcast_to(q_w.reshape(H, 1), (H, H)),
      jnp.broadcast_to(q_b.reshape(1, 1), (1, H)), W3_w, W3_b.reshape(1, H))

    n_vblocks = pl.cdiv(vocab, CW)
    all_scores, y_hat = pl.pallas_call(
        _score_kernel,
        grid=(n_vblocks,),
        in_specs=[
            pl.BlockSpec((B, H), lambda j: (0, 0)),
            pl.BlockSpec((CW, H), lambda j: (j, 0)),
            pl.BlockSpec((B, 1), lambda j: (0, 0)),
        ],
        out_specs=[
            pl.BlockSpec((B, CW), lambda j: (0, j)),
            pl.BlockSpec((B, 1), lambda j: (0, 0)),
        ],
        out_shape=[
            jax.ShapeDtypeStruct((B, vocab), jnp.float32),
            jax.ShapeDtypeStruct((B, 1), jnp.float32),
        ],
    )(s_h, embedding_table_weight, cue2)

    return (y_hat.reshape(B), all_scores)


# speedup vs baseline: 4.6551x; 1.0032x over previous
import jax
import jax.numpy as jnp
from jax import lax
from jax.experimental import pallas as pl

B = 1024
N_PER = 32
H = 128
L = 16
PAD = 8
SB = 64
SUB = 8
R = SB * L
RS = SUB * L
NS = SUB * N_PER
CW = 4096


def _attn_kernel(nodes_ref, seq_ref, il_ref, w1_ref, b1_ref, w2_ref, b2_ref,
                 q_ref, qb_ref, w3_ref, b3_ref, sh_ref):
    il = il_ref[...]

    base = (lax.broadcasted_iota(jnp.int32, (RS, NS), 0) >> 4) << 5
    cols = lax.broadcasted_iota(jnp.int32, (RS, NS), 1)
    parts = []
    for g in range(SB // SUB):
        seq_g = seq_ref[g * RS:(g + 1) * RS, :]
        counts = jnp.zeros((RS, NS), jnp.float32)
        for p in range(PAD):
            sp = seq_g[:, p:p + 1]
            t = jnp.where(sp < N_PER, sp, 100000) + base
            counts = counts + (cols == t).astype(jnp.float32)
        parts.append(jnp.dot(counts, nodes_ref[g * NS:(g + 1) * NS, :],
                             preferred_element_type=jnp.float32))
    sess_sum = jnp.concatenate(parts, axis=0)
    sess = sess_sum / il

    sel_r = lax.broadcasted_iota(jnp.int32, (SB, R), 1)
    sel_s = lax.broadcasted_iota(jnp.int32, (SB, R), 0) * L + (L - 1)
    sel = (sel_r == sel_s).astype(jnp.float32)
    v_n = jnp.dot(sel, sess, preferred_element_type=jnp.float32)

    rep = ((lax.broadcasted_iota(jnp.int32, (R, SB), 0) >> 4) ==
           lax.broadcasted_iota(jnp.int32, (R, SB), 1)).astype(jnp.float32)
    u1 = lax.dot_general(v_n, w1_ref[...], (((1,), (1,)), ((), ())),
                         preferred_element_type=jnp.float32) + b1_ref[...]
    t1 = jnp.dot(rep, u1, preferred_element_type=jnp.float32)
    t2 = lax.dot_general(sess, w2_ref[...], (((1,), (1,)), ((), ())),
                         preferred_element_type=jnp.float32) + b2_ref[...]
    a = jax.nn.sigmoid(t1 + t2)
    alpha_b = lax.dot_general(a, q_ref[...], (((1,), (0,)), ((), ())),
                              preferred_element_type=jnp.float32) + qb_ref[...]
    s_g = lax.dot_general(rep, alpha_b * sess, (((0,), (0,)), ((), ())),
                          preferred_element_type=jnp.float32)
    w3 = w3_ref[...]
    s_h = (lax.dot_general(v_n, w3[:, :H], (((1,), (1,)), ((), ())),
                           preferred_element_type=jnp.float32) +
           lax.dot_general(s_g, w3[:, H:], (((1,), (1,)), ((), ())),
                           preferred_element_type=jnp.float32) + b3_ref[...])
    sh_ref[...] = s_h


def _score_kernel(sh_ref, e_ref, cue_ref, out_ref, y_ref):
    j = pl.program_id(0)
    sh = sh_ref[...]
    eb = e_ref[...]
    scores = lax.dot_general(sh, eb, (((1,), (1,)), ((), ())),
                             preferred_element_type=jnp.float32)
    out_ref[...] = scores
    col = lax.broadcasted_iota(jnp.int32, (B, CW), 1) + j * CW
    hit = jnp.where(col == cue_ref[...], scores, 0.0)
    y_part = jnp.sum(hit, axis=1, keepdims=True)

    @pl.when(j == 0)
    def _():
        y_ref[...] = jnp.zeros_like(y_ref)

    y_ref[...] += y_part


def kernel(node_embedding, embedding_table_weight, batch, sequence, itemset_len,
           sequence_len, cue, W1_w, W1_b, W2_w, W2_b, q_w, q_b, W3_w, W3_b):
    vocab = embedding_table_weight.shape[0]
    il_f = itemset_len.astype(jnp.float32).reshape(B * L, 1)
    cue2 = cue.reshape(B, 1)

    n_blocks = B // SB
    s_h = pl.pallas_call(
        _attn_kernel,
        grid=(n_blocks,),
        in_specs=[
            pl.BlockSpec((SB * N_PER, H), lambda i: (i, 0)),
            pl.BlockSpec((R, PAD), lambda i: (i, 0)),
            pl.BlockSpec((R, 1), lambda i: (i, 0)),
            pl.BlockSpec((H, H), lambda i: (0, 0)),
            pl.BlockSpec((1, H), lambda i: (0, 0)),
            pl.BlockSpec((H, H), lambda i: (0, 0)),
            pl.BlockSpec((1, H), lambda i: (0, 0)),
            pl.BlockSpec((H, H), lambda i: (0, 0)),
            pl.BlockSpec((1, H), lambda i: (0, 0)),
            pl.BlockSpec((H, 2 * H), lambda i: (0, 0)),
            pl.BlockSpec((1, H), lambda i: (0, 0)),
        ],
        out_specs=pl.BlockSpec((SB, H), lambda i: (i, 0)),
        out_shape=jax.ShapeDtypeStruct((B, H), jnp.float32),
    )(node_embedding, sequence, il_f, W1_w, W1_b.reshape(1, H), W2_w,
      W2_b.reshape(1, H), jnp.broadcast_to(q_w.reshape(H, 1), (H, H)),
      jnp.broadcast_to(q_b.reshape(1, 1), (1, H)), W3_w, W3_b.reshape(1, H))

    n_vblocks = pl.cdiv(vocab, CW)
    all_scores, y_hat = pl.pallas_call(
        _score_kernel,
        grid=(n_vblocks,),
        in_specs=[
            pl.BlockSpec((B, H), lambda j: (0, 0)),
            pl.BlockSpec((CW, H), lambda j: (j, 0)),
            pl.BlockSpec((B, 1), lambda j: (0, 0)),
        ],
        out_specs=[
            pl.BlockSpec((B, CW), lambda j: (0, j)),
            pl.BlockSpec((B, 1), lambda j: (0, 0)),
        ],
        out_shape=[
            jax.ShapeDtypeStruct((B, vocab), jnp.float32),
            jax.ShapeDtypeStruct((B, 1), jnp.float32),
        ],
    )(s_h, embedding_table_weight, cue2)

    return (y_hat.reshape(B), all_scores)
